# Initial kernel scaffold; baseline (speedup 1.0000x reference)
import jax
import jax.numpy as jnp
from jax.experimental import pallas as pl


def kernel(obs, h_actor, h_critic, mask, cue, Wi_a, Wh_a, bi_a, bh_a, Wi_c, Wh_c, bi_c, bh_c, W_pol, b_pol, W_val, b_val, keys_a, vals_a, keys_c, vals_c):
    raise NotImplementedError("write your pallas kernel here")


# speedup vs baseline: 1.1149x; 1.1149x over previous
import functools

import jax
import jax.numpy as jnp
from jax import lax
from jax.experimental import pallas as pl
from jax.experimental.pallas import tpu as pltpu
from jax.experimental.pallas import tpu_sc as plsc

_B, _FEAT, _H, _A, _DICT = 64, 512, 128, 64, 100000
_KB = 2000
_NBLK = _DICT // _KB


def _scan_body(cue_ref, ka_ref, kc_ref, ia_ref, ic_ref, bva, bia, bvc, bic):
    step = pl.program_id(0)
    cue = cue_ref[...]

    def one(k_ref, bv, bi):
        kblk = k_ref[...]
        dots = lax.dot_general(cue, kblk, (((1,), (1,)), ((), ())),
                               preferred_element_type=jnp.float32)
        nk = jnp.sum(kblk * kblk, axis=1)
        score = dots - 0.5 * nk[None, :]
        bmax = jnp.max(score, axis=1, keepdims=True)
        barg = jnp.argmax(score, axis=1).astype(jnp.int32)[:, None] + step * _KB

        @pl.when(step == 0)
        def _():
            bv[...] = bmax
            bi[...] = barg

        @pl.when(step > 0)
        def _():
            upd = bmax > bv[...]
            bi[...] = jnp.where(upd, barg, bi[...])
            bv[...] = jnp.where(upd, bmax, bv[...])

    one(ka_ref, bva, bia)
    one(kc_ref, bvc, bic)

    @pl.when(step == _NBLK - 1)
    def _():
        ia_ref[...] = bia[...]
        ic_ref[...] = bic[...]


def _nn_indices(cue, keys_a, keys_c, interpret=False):
    return pl.pallas_call(
        _scan_body,
        grid=(_NBLK,),
        in_specs=[
            pl.BlockSpec((_B, _H), lambda i: (0, 0)),
            pl.BlockSpec((_KB, _H), lambda i: (i, 0)),
            pl.BlockSpec((_KB, _H), lambda i: (i, 0)),
        ],
        out_specs=[
            pl.BlockSpec((_B, 1), lambda i: (0, 0)),
            pl.BlockSpec((_B, 1), lambda i: (0, 0)),
        ],
        out_shape=[
            jax.ShapeDtypeStruct((_B, 1), jnp.int32),
            jax.ShapeDtypeStruct((_B, 1), jnp.int32),
        ],
        scratch_shapes=[
            pltpu.VMEM((_B, 1), jnp.float32),
            pltpu.VMEM((_B, 1), jnp.int32),
            pltpu.VMEM((_B, 1), jnp.float32),
            pltpu.VMEM((_B, 1), jnp.int32),
        ],
        compiler_params=pltpu.CompilerParams(
            dimension_semantics=("arbitrary",)),
        interpret=interpret,
    )(cue, keys_a, keys_c)


_ROWS_PER_W = 8


def _sc_gather(vals_a, idx_a, vals_c, idx_c):
    mesh = plsc.VectorSubcoreMesh(core_axis_name="c", subcore_axis_name="s")

    @functools.partial(
        pl.kernel,
        mesh=mesh,
        out_type=[
            jax.ShapeDtypeStruct((_B, _H), jnp.float32),
            jax.ShapeDtypeStruct((_B, _H), jnp.float32),
        ],
        scratch_types=[
            pltpu.VMEM((_ROWS_PER_W,), jnp.int32),
            pltpu.VMEM((_ROWS_PER_W, _H), jnp.float32),
            pltpu.SemaphoreType.DMA,
        ],
    )
    def gath(va_hbm, ia_hbm, vc_hbm, ic_hbm, ma_hbm, mc_hbm, idx_v, rows_v, sem):
        wid = lax.axis_index("s") * 2 + lax.axis_index("c")
        base = (wid % 8) * _ROWS_PER_W

        @pl.when(wid < 8)
        def _():
            pltpu.sync_copy(ia_hbm.at[pl.ds(base, _ROWS_PER_W)], idx_v)
            pltpu.async_copy(va_hbm.at[idx_v], rows_v, sem).wait()
            pltpu.sync_copy(rows_v, ma_hbm.at[pl.ds(base, _ROWS_PER_W)])

        @pl.when((wid >= 8) & (wid < 16))
        def _():
            pltpu.sync_copy(ic_hbm.at[pl.ds(base, _ROWS_PER_W)], idx_v)
            pltpu.async_copy(vc_hbm.at[idx_v], rows_v, sem).wait()
            pltpu.sync_copy(rows_v, mc_hbm.at[pl.ds(base, _ROWS_PER_W)])

    return gath(vals_a, idx_a, vals_c, idx_c)


def _tail_body(obs_ref, ha_ref, hc_ref, mask_ref, ma_ref, mc_ref,
               wia_ref, wha_ref, bia_ref, bha_ref,
               wic_ref, whc_ref, bic_ref, bhc_ref,
               wpol_ref, bpol_ref, wval_ref, bval_ref, gum_ref,
               act_ref, probs_ref, lp_ref, ent_ref, val_ref, hao_ref, hco_ref):
    x = obs_ref[...]

    def gru(h, m, Wi, Wh, bi, bh):
        gi = lax.dot_general(x, Wi, (((1,), (1,)), ((), ())),
                             preferred_element_type=jnp.float32) + bi
        gh = lax.dot_general(h, Wh, (((1,), (1,)), ((), ())),
                             preferred_element_type=jnp.float32) + bh
        i_r, i_z, i_n = gi[:, :_H], gi[:, _H:2 * _H], gi[:, 2 * _H:]
        h_r, h_z, h_n = gh[:, :_H], gh[:, _H:2 * _H], gh[:, 2 * _H:]
        r = jax.nn.sigmoid(i_r + h_r)
        z = jax.nn.sigmoid(i_z + h_z)
        n = jnp.tanh(i_n + r * (h_n + m))
        return (1.0 - z) * n + z * h

    hA = gru(ha_ref[...], ma_ref[...], wia_ref[...], wha_ref[...],
             bia_ref[...], bha_ref[...])
    hC = gru(hc_ref[...], mc_ref[...], wic_ref[...], whc_ref[...],
             bic_ref[...], bhc_ref[...])

    logits = lax.dot_general(hA, wpol_ref[...], (((1,), (1,)), ((), ())),
                             preferred_element_type=jnp.float32) + bpol_ref[...]
    valid = mask_ref[...] > 0
    neg = jnp.finfo(jnp.float32).min
    masked = jnp.where(valid, logits, neg)
    mx = jnp.max(masked, axis=1, keepdims=True)
    shifted = masked - mx
    ex = jnp.exp(shifted)
    denom = jnp.sum(ex, axis=1, keepdims=True)
    logp = shifted - jnp.log(denom)
    probs = ex / denom

    action = jnp.argmax(masked + gum_ref[...], axis=1).astype(jnp.int32)
    lanes = lax.broadcasted_iota(jnp.int32, (_B, _A), 1)
    onehot = lanes == action[:, None]
    log_prob = jnp.sum(jnp.where(onehot, logp, 0.0), axis=1, keepdims=True)
    entropy = -jnp.sum(jnp.where(valid, logp * probs, 0.0), axis=1, keepdims=True)
    value = jnp.sum(hC * wval_ref[...], axis=1, keepdims=True) + bval_ref[...]

    act_ref[...] = action[:, None]
    probs_ref[...] = probs
    lp_ref[...] = log_prob
    ent_ref[...] = entropy
    val_ref[...] = value
    hao_ref[...] = hA
    hco_ref[...] = hC


def _tail(obs, h_actor, h_critic, mask_i32, m_a, m_c,
          Wi_a, Wh_a, bi_a, bh_a, Wi_c, Wh_c, bi_c, bh_c,
          W_pol, b_pol, W_val, b_val, gum, interpret=False):
    return pl.pallas_call(
        _tail_body,
        out_shape=[
            jax.ShapeDtypeStruct((_B, 1), jnp.int32),
            jax.ShapeDtypeStruct((_B, _A), jnp.float32),
            jax.ShapeDtypeStruct((_B, 1), jnp.float32),
            jax.ShapeDtypeStruct((_B, 1), jnp.float32),
            jax.ShapeDtypeStruct((_B, 1), jnp.float32),
            jax.ShapeDtypeStruct((_B, _H), jnp.float32),
            jax.ShapeDtypeStruct((_B, _H), jnp.float32),
        ],
        interpret=interpret,
    )(obs, h_actor, h_critic, mask_i32, m_a, m_c,
      Wi_a, Wh_a, bi_a.reshape(1, -1), bh_a.reshape(1, -1),
      Wi_c, Wh_c, bi_c.reshape(1, -1), bh_c.reshape(1, -1),
      W_pol, b_pol.reshape(1, -1), W_val, b_val.reshape(1, 1), gum)


def kernel(obs, h_actor, h_critic, mask, cue,
           Wi_a, Wh_a, bi_a, bh_a, Wi_c, Wh_c, bi_c, bh_c,
           W_pol, b_pol, W_val, b_val, keys_a, vals_a, keys_c, vals_c):
    ia, ic = _nn_indices(cue, keys_a, keys_c)
    m_a, m_c = _sc_gather(vals_a, ia.reshape(_B), vals_c, ic.reshape(_B))
    gum = jax.random.gumbel(jax.random.key(42), (_B, _A), jnp.float32)
    act, probs, lp, ent, val, hA, hC = _tail(
        obs.reshape(_B, -1), h_actor, h_critic, mask.astype(jnp.int32),
        m_a, m_c, Wi_a, Wh_a, bi_a, bh_a, Wi_c, Wh_c, bi_c, bh_c,
        W_pol, b_pol, W_val, b_val, gum)
    return (act.reshape(_B), probs, lp.reshape(_B), ent.reshape(_B),
            val, hA, hC)


# speedup vs baseline: 1.2749x; 1.1435x over previous
import functools

import jax
import jax.numpy as jnp
from jax import lax
from jax.experimental import pallas as pl
from jax.experimental.pallas import tpu as pltpu
from jax.experimental.pallas import tpu_sc as plsc

_B, _FEAT, _H, _A, _DICT = 64, 512, 128, 64, 100000
_KB = 4000
_NBLK = _DICT // _KB


def _scan_body(cue_ref, ka_ref, kc_ref, ia_ref, ic_ref, bva, bia, bvc, bic):
    step = pl.program_id(0)
    cue = cue_ref[...]

    def one(k_ref, bv, bi):
        kblk = k_ref[...]
        dots = lax.dot_general(cue, kblk, (((1,), (1,)), ((), ())),
                               preferred_element_type=jnp.float32)
        nk = jnp.sum(kblk * kblk, axis=1)
        score = dots - 0.5 * nk[None, :]
        bmax = jnp.max(score, axis=1, keepdims=True)
        barg = jnp.argmax(score, axis=1).astype(jnp.int32)[:, None] + step * _KB

        @pl.when(step == 0)
        def _():
            bv[...] = bmax
            bi[...] = barg

        @pl.when(step > 0)
        def _():
            upd = bmax > bv[...]
            bi[...] = jnp.where(upd, barg, bi[...])
            bv[...] = jnp.where(upd, bmax, bv[...])

    one(ka_ref, bva, bia)
    one(kc_ref, bvc, bic)

    @pl.when(step == _NBLK - 1)
    def _():
        ia_ref[...] = bia[...]
        ic_ref[...] = bic[...]


def _nn_indices(cue, keys_a, keys_c, interpret=False):
    return pl.pallas_call(
        _scan_body,
        grid=(_NBLK,),
        in_specs=[
            pl.BlockSpec((_B, _H), lambda i: (0, 0)),
            pl.BlockSpec((_KB, _H), lambda i: (i, 0)),
            pl.BlockSpec((_KB, _H), lambda i: (i, 0)),
        ],
        out_specs=[
            pl.BlockSpec((_B, 1), lambda i: (0, 0)),
            pl.BlockSpec((_B, 1), lambda i: (0, 0)),
        ],
        out_shape=[
            jax.ShapeDtypeStruct((_B, 1), jnp.int32),
            jax.ShapeDtypeStruct((_B, 1), jnp.int32),
        ],
        scratch_shapes=[
            pltpu.VMEM((_B, 1), jnp.float32),
            pltpu.VMEM((_B, 1), jnp.int32),
            pltpu.VMEM((_B, 1), jnp.float32),
            pltpu.VMEM((_B, 1), jnp.int32),
        ],
        compiler_params=pltpu.CompilerParams(
            dimension_semantics=("arbitrary",)),
        interpret=interpret,
    )(cue, keys_a, keys_c)


_ROWS_PER_W = 8


def _sc_gather(vals_a, idx_a, vals_c, idx_c):
    mesh = plsc.VectorSubcoreMesh(core_axis_name="c", subcore_axis_name="s")

    @functools.partial(
        pl.kernel,
        mesh=mesh,
        out_type=[
            jax.ShapeDtypeStruct((_B, _H), jnp.float32),
            jax.ShapeDtypeStruct((_B, _H), jnp.float32),
        ],
        scratch_types=[
            pltpu.VMEM((_ROWS_PER_W,), jnp.int32),
            pltpu.VMEM((_ROWS_PER_W, _H), jnp.float32),
            pltpu.SemaphoreType.DMA,
        ],
    )
    def gath(va_hbm, ia_hbm, vc_hbm, ic_hbm, ma_hbm, mc_hbm, idx_v, rows_v, sem):
        wid = lax.axis_index("s") * 2 + lax.axis_index("c")
        base = (wid % 8) * _ROWS_PER_W

        @pl.when(wid < 8)
        def _():
            pltpu.sync_copy(ia_hbm.at[pl.ds(base, _ROWS_PER_W)], idx_v)
            pltpu.async_copy(va_hbm.at[idx_v], rows_v, sem).wait()
            pltpu.sync_copy(rows_v, ma_hbm.at[pl.ds(base, _ROWS_PER_W)])

        @pl.when((wid >= 8) & (wid < 16))
        def _():
            pltpu.sync_copy(ic_hbm.at[pl.ds(base, _ROWS_PER_W)], idx_v)
            pltpu.async_copy(vc_hbm.at[idx_v], rows_v, sem).wait()
            pltpu.sync_copy(rows_v, mc_hbm.at[pl.ds(base, _ROWS_PER_W)])

    return gath(vals_a, idx_a, vals_c, idx_c)


def _tail_body(obs_ref, ha_ref, hc_ref, mask_ref, ma_ref, mc_ref,
               wia_ref, wha_ref, bia_ref, bha_ref,
               wic_ref, whc_ref, bic_ref, bhc_ref,
               wpol_ref, bpol_ref, wval_ref, bval_ref, gum_ref,
               act_ref, probs_ref, lp_ref, ent_ref, val_ref, hao_ref, hco_ref):
    x = obs_ref[...]

    def gru(h, m, Wi, Wh, bi, bh):
        gi = lax.dot_general(x, Wi, (((1,), (1,)), ((), ())),
                             preferred_element_type=jnp.float32) + bi
        gh = lax.dot_general(h, Wh, (((1,), (1,)), ((), ())),
                             preferred_element_type=jnp.float32) + bh
        i_r, i_z, i_n = gi[:, :_H], gi[:, _H:2 * _H], gi[:, 2 * _H:]
        h_r, h_z, h_n = gh[:, :_H], gh[:, _H:2 * _H], gh[:, 2 * _H:]
        r = jax.nn.sigmoid(i_r + h_r)
        z = jax.nn.sigmoid(i_z + h_z)
        n = jnp.tanh(i_n + r * (h_n + m))
        return (1.0 - z) * n + z * h

    hA = gru(ha_ref[...], ma_ref[...], wia_ref[...], wha_ref[...],
             bia_ref[...], bha_ref[...])
    hC = gru(hc_ref[...], mc_ref[...], wic_ref[...], whc_ref[...],
             bic_ref[...], bhc_ref[...])

    logits = lax.dot_general(hA, wpol_ref[...], (((1,), (1,)), ((), ())),
                             preferred_element_type=jnp.float32) + bpol_ref[...]
    valid = mask_ref[...] > 0
    neg = jnp.finfo(jnp.float32).min
    masked = jnp.where(valid, logits, neg)
    mx = jnp.max(masked, axis=1, keepdims=True)
    shifted = masked - mx
    ex = jnp.exp(shifted)
    denom = jnp.sum(ex, axis=1, keepdims=True)
    logp = shifted - jnp.log(denom)
    probs = ex / denom

    action = jnp.argmax(masked + gum_ref[...], axis=1).astype(jnp.int32)
    lanes = lax.broadcasted_iota(jnp.int32, (_B, _A), 1)
    onehot = lanes == action[:, None]
    log_prob = jnp.sum(jnp.where(onehot, logp, 0.0), axis=1, keepdims=True)
    entropy = -jnp.sum(jnp.where(valid, logp * probs, 0.0), axis=1, keepdims=True)
    value = jnp.sum(hC * wval_ref[...], axis=1, keepdims=True) + bval_ref[...]

    act_ref[...] = action[:, None]
    probs_ref[...] = probs
    lp_ref[...] = log_prob
    ent_ref[...] = entropy
    val_ref[...] = value
    hao_ref[...] = hA
    hco_ref[...] = hC


def _tail(obs, h_actor, h_critic, mask_i32, m_a, m_c,
          Wi_a, Wh_a, bi_a, bh_a, Wi_c, Wh_c, bi_c, bh_c,
          W_pol, b_pol, W_val, b_val, gum, interpret=False):
    return pl.pallas_call(
        _tail_body,
        out_shape=[
            jax.ShapeDtypeStruct((_B, 1), jnp.int32),
            jax.ShapeDtypeStruct((_B, _A), jnp.float32),
            jax.ShapeDtypeStruct((_B, 1), jnp.float32),
            jax.ShapeDtypeStruct((_B, 1), jnp.float32),
            jax.ShapeDtypeStruct((_B, 1), jnp.float32),
            jax.ShapeDtypeStruct((_B, _H), jnp.float32),
            jax.ShapeDtypeStruct((_B, _H), jnp.float32),
        ],
        interpret=interpret,
    )(obs, h_actor, h_critic, mask_i32, m_a, m_c,
      Wi_a, Wh_a, bi_a.reshape(1, -1), bh_a.reshape(1, -1),
      Wi_c, Wh_c, bi_c.reshape(1, -1), bh_c.reshape(1, -1),
      W_pol, b_pol.reshape(1, -1), W_val, b_val.reshape(1, 1), gum)


def kernel(obs, h_actor, h_critic, mask, cue,
           Wi_a, Wh_a, bi_a, bh_a, Wi_c, Wh_c, bi_c, bh_c,
           W_pol, b_pol, W_val, b_val, keys_a, vals_a, keys_c, vals_c):
    ia, ic = _nn_indices(cue, keys_a, keys_c)
    m_a, m_c = _sc_gather(vals_a, ia.reshape(_B), vals_c, ic.reshape(_B))
    gum = jax.random.gumbel(jax.random.key(42), (_B, _A), jnp.float32)
    act, probs, lp, ent, val, hA, hC = _tail(
        obs.reshape(_B, -1), h_actor, h_critic, mask.astype(jnp.int32),
        m_a, m_c, Wi_a, Wh_a, bi_a, bh_a, Wi_c, Wh_c, bi_c, bh_c,
        W_pol, b_pol, W_val, b_val, gum)
    return (act.reshape(_B), probs, lp.reshape(_B), ent.reshape(_B),
            val, hA, hC)


# speedup vs baseline: 1.4517x; 1.1387x over previous
import functools

import jax
import jax.numpy as jnp
from jax import lax
from jax.experimental import pallas as pl
from jax.experimental.pallas import tpu as pltpu
from jax.experimental.pallas import tpu_sc as plsc

_B, _FEAT, _H, _A, _DICT = 64, 512, 128, 64, 100000
_KB = 10000
_NBLK = _DICT // _KB


def _scan_body(cue_ref, ka_ref, kc_ref, ia_ref, ic_ref, bva, bia, bvc, bic):
    step = pl.program_id(0)
    cue = cue_ref[...]

    def one(k_ref, bv, bi):
        kblk = k_ref[...]
        dots = lax.dot_general(cue, kblk, (((1,), (1,)), ((), ())),
                               preferred_element_type=jnp.float32)
        nk = jnp.sum(kblk * kblk, axis=1)
        score = dots - 0.5 * nk[None, :]
        bmax = jnp.max(score, axis=1, keepdims=True)
        barg = jnp.argmax(score, axis=1).astype(jnp.int32)[:, None] + step * _KB

        @pl.when(step == 0)
        def _():
            bv[...] = bmax
            bi[...] = barg

        @pl.when(step > 0)
        def _():
            upd = bmax > bv[...]
            bi[...] = jnp.where(upd, barg, bi[...])
            bv[...] = jnp.where(upd, bmax, bv[...])

    one(ka_ref, bva, bia)
    one(kc_ref, bvc, bic)

    @pl.when(step == _NBLK - 1)
    def _():
        ia_ref[...] = bia[...]
        ic_ref[...] = bic[...]


def _nn_indices(cue, keys_a, keys_c, interpret=False):
    return pl.pallas_call(
        _scan_body,
        grid=(_NBLK,),
        in_specs=[
            pl.BlockSpec((_B, _H), lambda i: (0, 0)),
            pl.BlockSpec((_KB, _H), lambda i: (i, 0)),
            pl.BlockSpec((_KB, _H), lambda i: (i, 0)),
        ],
        out_specs=[
            pl.BlockSpec((_B, 1), lambda i: (0, 0)),
            pl.BlockSpec((_B, 1), lambda i: (0, 0)),
        ],
        out_shape=[
            jax.ShapeDtypeStruct((_B, 1), jnp.int32),
            jax.ShapeDtypeStruct((_B, 1), jnp.int32),
        ],
        scratch_shapes=[
            pltpu.VMEM((_B, 1), jnp.float32),
            pltpu.VMEM((_B, 1), jnp.int32),
            pltpu.VMEM((_B, 1), jnp.float32),
            pltpu.VMEM((_B, 1), jnp.int32),
        ],
        compiler_params=pltpu.CompilerParams(
            dimension_semantics=("arbitrary",)),
        interpret=interpret,
    )(cue, keys_a, keys_c)


_ROWS_PER_W = 8


def _sc_gather(vals_a, idx_a, vals_c, idx_c):
    mesh = plsc.VectorSubcoreMesh(core_axis_name="c", subcore_axis_name="s")

    @functools.partial(
        pl.kernel,
        mesh=mesh,
        out_type=[
            jax.ShapeDtypeStruct((_B, _H), jnp.float32),
            jax.ShapeDtypeStruct((_B, _H), jnp.float32),
        ],
        scratch_types=[
            pltpu.VMEM((_ROWS_PER_W,), jnp.int32),
            pltpu.VMEM((_ROWS_PER_W, _H), jnp.float32),
            pltpu.SemaphoreType.DMA,
        ],
    )
    def gath(va_hbm, ia_hbm, vc_hbm, ic_hbm, ma_hbm, mc_hbm, idx_v, rows_v, sem):
        wid = lax.axis_index("s") * 2 + lax.axis_index("c")
        base = (wid % 8) * _ROWS_PER_W

        @pl.when(wid < 8)
        def _():
            pltpu.sync_copy(ia_hbm.at[pl.ds(base, _ROWS_PER_W)], idx_v)
            pltpu.async_copy(va_hbm.at[idx_v], rows_v, sem).wait()
            pltpu.sync_copy(rows_v, ma_hbm.at[pl.ds(base, _ROWS_PER_W)])

        @pl.when((wid >= 8) & (wid < 16))
        def _():
            pltpu.sync_copy(ic_hbm.at[pl.ds(base, _ROWS_PER_W)], idx_v)
            pltpu.async_copy(vc_hbm.at[idx_v], rows_v, sem).wait()
            pltpu.sync_copy(rows_v, mc_hbm.at[pl.ds(base, _ROWS_PER_W)])

    return gath(vals_a, idx_a, vals_c, idx_c)


def _tail_body(obs_ref, ha_ref, hc_ref, mask_ref, ma_ref, mc_ref,
               wia_ref, wha_ref, bia_ref, bha_ref,
               wic_ref, whc_ref, bic_ref, bhc_ref,
               wpol_ref, bpol_ref, wval_ref, bval_ref, gum_ref,
               act_ref, probs_ref, lp_ref, ent_ref, val_ref, hao_ref, hco_ref):
    x = obs_ref[...]

    def gru(h, m, Wi, Wh, bi, bh):
        gi = lax.dot_general(x, Wi, (((1,), (1,)), ((), ())),
                             preferred_element_type=jnp.float32) + bi
        gh = lax.dot_general(h, Wh, (((1,), (1,)), ((), ())),
                             preferred_element_type=jnp.float32) + bh
        i_r, i_z, i_n = gi[:, :_H], gi[:, _H:2 * _H], gi[:, 2 * _H:]
        h_r, h_z, h_n = gh[:, :_H], gh[:, _H:2 * _H], gh[:, 2 * _H:]
        r = jax.nn.sigmoid(i_r + h_r)
        z = jax.nn.sigmoid(i_z + h_z)
        n = jnp.tanh(i_n + r * (h_n + m))
        return (1.0 - z) * n + z * h

    hA = gru(ha_ref[...], ma_ref[...], wia_ref[...], wha_ref[...],
             bia_ref[...], bha_ref[...])
    hC = gru(hc_ref[...], mc_ref[...], wic_ref[...], whc_ref[...],
             bic_ref[...], bhc_ref[...])

    logits = lax.dot_general(hA, wpol_ref[...], (((1,), (1,)), ((), ())),
                             preferred_element_type=jnp.float32) + bpol_ref[...]
    valid = mask_ref[...] > 0
    neg = jnp.finfo(jnp.float32).min
    masked = jnp.where(valid, logits, neg)
    mx = jnp.max(masked, axis=1, keepdims=True)
    shifted = masked - mx
    ex = jnp.exp(shifted)
    denom = jnp.sum(ex, axis=1, keepdims=True)
    logp = shifted - jnp.log(denom)
    probs = ex / denom

    action = jnp.argmax(masked + gum_ref[...], axis=1).astype(jnp.int32)
    lanes = lax.broadcasted_iota(jnp.int32, (_B, _A), 1)
    onehot = lanes == action[:, None]
    log_prob = jnp.sum(jnp.where(onehot, logp, 0.0), axis=1, keepdims=True)
    entropy = -jnp.sum(jnp.where(valid, logp * probs, 0.0), axis=1, keepdims=True)
    value = jnp.sum(hC * wval_ref[...], axis=1, keepdims=True) + bval_ref[...]

    act_ref[...] = action[:, None]
    probs_ref[...] = probs
    lp_ref[...] = log_prob
    ent_ref[...] = entropy
    val_ref[...] = value
    hao_ref[...] = hA
    hco_ref[...] = hC


def _tail(obs, h_actor, h_critic, mask_i32, m_a, m_c,
          Wi_a, Wh_a, bi_a, bh_a, Wi_c, Wh_c, bi_c, bh_c,
          W_pol, b_pol, W_val, b_val, gum, interpret=False):
    return pl.pallas_call(
        _tail_body,
        out_shape=[
            jax.ShapeDtypeStruct((_B, 1), jnp.int32),
            jax.ShapeDtypeStruct((_B, _A), jnp.float32),
            jax.ShapeDtypeStruct((_B, 1), jnp.float32),
            jax.ShapeDtypeStruct((_B, 1), jnp.float32),
            jax.ShapeDtypeStruct((_B, 1), jnp.float32),
            jax.ShapeDtypeStruct((_B, _H), jnp.float32),
            jax.ShapeDtypeStruct((_B, _H), jnp.float32),
        ],
        interpret=interpret,
    )(obs, h_actor, h_critic, mask_i32, m_a, m_c,
      Wi_a, Wh_a, bi_a.reshape(1, -1), bh_a.reshape(1, -1),
      Wi_c, Wh_c, bi_c.reshape(1, -1), bh_c.reshape(1, -1),
      W_pol, b_pol.reshape(1, -1), W_val, b_val.reshape(1, 1), gum)


def kernel(obs, h_actor, h_critic, mask, cue,
           Wi_a, Wh_a, bi_a, bh_a, Wi_c, Wh_c, bi_c, bh_c,
           W_pol, b_pol, W_val, b_val, keys_a, vals_a, keys_c, vals_c):
    ia, ic = _nn_indices(cue, keys_a, keys_c)
    m_a, m_c = _sc_gather(vals_a, ia.reshape(_B), vals_c, ic.reshape(_B))
    gum = jax.random.gumbel(jax.random.key(42), (_B, _A), jnp.float32)
    act, probs, lp, ent, val, hA, hC = _tail(
        obs.reshape(_B, -1), h_actor, h_critic, mask.astype(jnp.int32),
        m_a, m_c, Wi_a, Wh_a, bi_a, bh_a, Wi_c, Wh_c, bi_c, bh_c,
        W_pol, b_pol, W_val, b_val, gum)
    return (act.reshape(_B), probs, lp.reshape(_B), ent.reshape(_B),
            val, hA, hC)
